# Initial kernel scaffold; baseline (speedup 1.0000x reference)
import jax
import jax.numpy as jnp
from jax.experimental import pallas as pl


def kernel(users, items, edge_index, edge_values, user_emb, item_emb):
    raise NotImplementedError("write your pallas kernel here")


# speedup vs baseline: 3.8803x; 3.8803x over previous
import functools
import jax
import jax.numpy as jnp
from jax import lax
from jax.experimental import pallas as pl
from jax.experimental.pallas import tpu as pltpu
from jax.experimental.pallas import tpu_sc as plsc

NC = 2
NS = 16
L = 16

SUB = 80
SUBS_PER_BIG = 5
BIG = SUB * SUBS_PER_BIG

ACC_PAD_ROWS = 25088


def _propagate_body(n_half, bigs_per_tile, emb_in, src1, dst1, w1, emb_out,
                    srcb, dstb, wb, dst2, rows0, rows1, acc, gsem, sem):
    c = lax.axis_index("c")
    s = lax.axis_index("s")
    base = c * n_half
    trash = jnp.int32(n_half)

    zvec = jnp.zeros((L,), jnp.float32)

    def zb_body(i, carry):
        for r in range(4):
            rows0[i, pl.ds(r * L, L)] = zvec
        return carry
    lax.fori_loop(0, SUB, zb_body, None)

    rows_per_tile = ACC_PAD_ROWS // NS
    for k in range(rows_per_tile // SUB):
        pltpu.sync_copy(rows0, acc.at[pl.ds(s * rows_per_tile + k * SUB, SUB)])
    zrem = rows_per_tile % SUB
    if zrem:
        pltpu.sync_copy(
            rows0.at[pl.ds(0, zrem)],
            acc.at[pl.ds(s * rows_per_tile + rows_per_tile - zrem, zrem)])
    plsc.subcore_barrier()

    rows_bufs = (rows0, rows1)
    sems = (sem, gsem)
    edges_per_tile = bigs_per_tile * BIG

    def big_body(g, carry):
        e_off = s * edges_per_tile + g * BIG
        pltpu.sync_copy(src1.at[pl.ds(e_off, BIG)], srcb)
        pltpu.sync_copy(dst1.at[pl.ds(e_off, BIG)], dstb)
        pltpu.sync_copy(w1.at[pl.ds(e_off, BIG)], wb)

        for j in range(SUBS_PER_BIG):
            for q in range(SUB // L):
                v = dstb[pl.ds(j * SUB + q * L, L)]
                ok = (v >= base) & (v < base + n_half)
                dst2[j, pl.ds(q * L, L)] = jnp.where(ok, v - base, trash)

        copies = [pltpu.async_copy(
            emb_in.at[srcb.at[pl.ds(0, SUB)]], rows_bufs[0], sems[0])]
        for j in range(SUBS_PER_BIG):
            p = j % 2
            if j + 1 < SUBS_PER_BIG:
                copies.append(pltpu.async_copy(
                    emb_in.at[srcb.at[pl.ds((j + 1) * SUB, SUB)]],
                    rows_bufs[1 - p], sems[1 - p]))
            copies[j].wait()
            rows = rows_bufs[p]

            def scale_body(e, carry2):
                wv = plsc.load_gather(
                    wb, [jnp.full((L,), j * SUB, jnp.int32) + e])
                for r in range(4):
                    rows[e, pl.ds(r * L, L)] = rows[e, pl.ds(r * L, L)] * wv
                return carry2
            lax.fori_loop(0, SUB, scale_body, None)
            pltpu.sync_copy(rows, acc.at[dst2.at[j]], add=True)
        return carry

    lax.fori_loop(0, bigs_per_tile, big_body, None)
    plsc.subcore_barrier()

    wb_rows = 1560
    pltpu.sync_copy(acc.at[pl.ds(s * wb_rows, wb_rows)],
                    emb_out.at[pl.ds(base + s * wb_rows, wb_rows)])
    rem = n_half - NS * wb_rows
    if rem:
        @pl.when(s == NS - 1)
        def _tail():
            pltpu.sync_copy(acc.at[pl.ds(NS * wb_rows, rem)],
                            emb_out.at[pl.ds(base + NS * wb_rows, rem)])


def _make_propagate(n, d, e):
    n_half = n // NC
    edges_per_tile = e // NS
    bigs_per_tile = edges_per_tile // BIG
    assert edges_per_tile % BIG == 0 and d == 64
    zrows = (ACC_PAD_ROWS // NS) // 2
    mesh = plsc.VectorSubcoreMesh(core_axis_name="c", subcore_axis_name="s")
    return pl.kernel(
        functools.partial(_propagate_body, n_half, bigs_per_tile),
        out_type=jax.ShapeDtypeStruct((n, d), jnp.float32),
        mesh=mesh,
        scratch_types=[
            pltpu.VMEM((BIG,), jnp.int32),
            pltpu.VMEM((BIG,), jnp.int32),
            pltpu.VMEM((BIG,), jnp.float32),
            pltpu.VMEM((SUBS_PER_BIG, SUB), jnp.int32),
            pltpu.VMEM((SUB, 64), jnp.float32),
            pltpu.VMEM((SUB, 64), jnp.float32),
            pltpu.VMEM_SHARED((ACC_PAD_ROWS, 64), jnp.float32),
            pltpu.SemaphoreType.DMA,
            pltpu.SemaphoreType.DMA,
        ],
        compiler_params=pltpu.CompilerParams(needs_layout_passes=False, use_tc_tiling_on_sc=False),
        name="lightgcn_propagate",
    )


def _final_body(n_users, bpt, users, items, emb0, emb3, gamma,
                ub, ib, u0r, u3r, i0r, i3r, gb, sem):
    c = lax.axis_index("c")
    s = lax.axis_index("s")
    wid = s * NC + c
    b0 = wid * bpt

    pltpu.sync_copy(users.at[pl.ds(b0, bpt)], ub)
    pltpu.sync_copy(items.at[pl.ds(b0, bpt)], ib)
    for q in range(bpt // L):
        ib[pl.ds(q * L, L)] = ib[pl.ds(q * L, L)] + jnp.int32(n_users)

    pltpu.async_copy(emb0.at[ub], u0r, sem).wait()
    pltpu.async_copy(emb3.at[ub], u3r, sem).wait()
    pltpu.async_copy(emb0.at[ib], i0r, sem).wait()
    pltpu.async_copy(emb3.at[ib], i3r, sem).wait()

    iota = lax.iota(jnp.int32, L)
    for q in range(bpt // L):
        bvec = iota + jnp.int32(q * L)

        def dot_body(dd, acc):
            dvec = jnp.full((L,), dd, jnp.int32)
            u0 = plsc.load_gather(u0r, [bvec, dvec])
            u3 = plsc.load_gather(u3r, [bvec, dvec])
            i0 = plsc.load_gather(i0r, [bvec, dvec])
            i3 = plsc.load_gather(i3r, [bvec, dvec])
            return acc + (u0 + u3) * (i0 + i3)
        acc = lax.fori_loop(0, 64, dot_body, jnp.zeros((L,), jnp.float32))
        gb[pl.ds(q * L, L)] = acc * 0.25

    pltpu.sync_copy(gb, gamma.at[pl.ds(b0, bpt)])


def _make_final(n_users, b):
    bpt = b // (NC * NS)
    mesh = plsc.VectorSubcoreMesh(core_axis_name="c", subcore_axis_name="s")
    return pl.kernel(
        functools.partial(_final_body, n_users, bpt),
        out_type=jax.ShapeDtypeStruct((b,), jnp.float32),
        mesh=mesh,
        scratch_types=[
            pltpu.VMEM((bpt,), jnp.int32),
            pltpu.VMEM((bpt,), jnp.int32),
            pltpu.VMEM((bpt, 64), jnp.float32),
            pltpu.VMEM((bpt, 64), jnp.float32),
            pltpu.VMEM((bpt, 64), jnp.float32),
            pltpu.VMEM((bpt, 64), jnp.float32),
            pltpu.VMEM((bpt,), jnp.float32),
            pltpu.SemaphoreType.DMA,
        ],
        compiler_params=pltpu.CompilerParams(needs_layout_passes=False, use_tc_tiling_on_sc=False),
        name="lightgcn_final",
    )


@jax.jit
def kernel(users, items, edge_index, edge_values, user_emb, item_emb):
    n_users, d = user_emb.shape
    n = n_users + item_emb.shape[0]
    e = edge_values.shape[0]
    b = users.shape[0]

    src1 = edge_index[0]
    dst1 = edge_index[1]
    emb0 = jnp.concatenate([user_emb, item_emb], axis=0)

    propagate = _make_propagate(n, d, e)
    emb = emb0
    for _ in range(3):
        emb = propagate(emb, src1, dst1, edge_values)
    return _make_final(n_users, b)(users, items, emb0, emb)
